# baseline (speedup 1.0000x reference)
import functools

import jax
import jax.numpy as jnp
from jax import lax
from jax.experimental import pallas as pl
from jax.experimental.pallas import tpu as pltpu
from jax.experimental.pallas import tpu_sc as plsc

NUM_CLIENTS = 1024
NUM_CLUSTERS = 64
PCA_DIM = 4
SINKHORN_REG = 0.2
TEMPERATURE = 0.5

NUM_SC_CORES = 2
NUM_SC_SUBCORES = 16
LANES = 16
NUM_WORKERS = NUM_SC_CORES * NUM_SC_SUBCORES
ROWS_PER_WORKER = NUM_CLIENTS // NUM_WORKERS
CHUNKS = NUM_CLUSTERS // LANES


def _sqrt16(x):
    y = 0.5 * (1.0 + x)
    for _ in range(16):
        y = 0.5 * (y + x / y)
    return jnp.where(x > 0, y, 0.0)


def _sc_hist_body(ct_hbm, ltr_hbm, out_hbm, ct_v, lg_v, cn_v, cnt_v):
    wid = lax.axis_index("s") * NUM_SC_CORES + lax.axis_index("c")
    pltpu.sync_copy(ct_hbm, ct_v)
    pltpu.sync_copy(ltr_hbm.at[wid], lg_v)

    for c in range(CHUNKS):
        cn2 = None
        for j in range(PCA_DIM):
            row = ct_v[j, pl.ds(c * LANES, LANES)]
            sq = row * row
            cn2 = sq if cn2 is None else cn2 + sq
        cn_v[pl.ds(c * LANES, LANES)] = 2.0 * _sqrt16(cn2)

    neg = jnp.full((LANES,), -3.4e38, jnp.float32)
    ms = [neg, neg]
    idxs = [jnp.zeros((LANES,), jnp.int32) for _ in range(2)]
    for k in range(NUM_CLUSTERS):
        bk = plsc.load_gather(cn_v, [jnp.full((LANES,), k, jnp.int32)])
        for g in range(2):
            v = lg_v[k, pl.ds(g * LANES, LANES)] - bk
            better = v > ms[g]
            ms[g] = jnp.where(better, v, ms[g])
            idxs[g] = jnp.where(better, jnp.int32(k), idxs[g])

    for c in range(CHUNKS):
        cnt_v[pl.ds(c * LANES, LANES)] = jnp.zeros((LANES,), jnp.float32)
    ones = jnp.ones((LANES,), jnp.float32)
    for g in range(2):
        plsc.addupdate_scatter(cnt_v, [idxs[g]], ones)
    pltpu.sync_copy(cnt_v, out_hbm.at[wid])


_sc_hist = functools.partial(
    pl.kernel,
    out_type=jax.ShapeDtypeStruct((NUM_WORKERS, NUM_CLUSTERS), jnp.float32),
    mesh=plsc.VectorSubcoreMesh(
        core_axis_name="c", subcore_axis_name="s",
        num_cores=NUM_SC_CORES, num_subcores=NUM_SC_SUBCORES),
    compiler_params=pltpu.CompilerParams(needs_layout_passes=False),
    scratch_types=[
        pltpu.VMEM((PCA_DIM, NUM_CLUSTERS), jnp.float32),
        pltpu.VMEM((NUM_CLUSTERS, ROWS_PER_WORKER), jnp.float32),
        pltpu.VMEM((NUM_CLUSTERS,), jnp.float32),
        pltpu.VMEM((NUM_CLUSTERS,), jnp.float32),
    ],
)(_sc_hist_body)


def _vwc_body(centers_ref, ct_ref, logits_ref, partials_ref, probs_ref,
              loss_ref):
    c = centers_ref[...]
    ct = ct_ref[...]
    lg = logits_ref[...]

    cn2_row = jnp.sum(ct * ct, axis=0, keepdims=True)
    cn_row = jnp.where(cn2_row > 0,
                       jnp.sqrt(jnp.where(cn2_row > 0, cn2_row, 1.0)), 0.0)

    a = lg - (1.0 / TEMPERATURE) * cn_row
    m = jnp.max(a, axis=1, keepdims=True)
    e = jnp.exp(a - m)
    s = jnp.sum(e, axis=1, keepdims=True)
    probs = e / s
    probs_ref[...] = probs

    colsum = jnp.sum(probs, axis=0, keepdims=True)
    cluster_probs = colsum * (1.0 / NUM_CLIENTS)
    entropy = -jnp.sum(cluster_probs * jnp.log(cluster_probs + 1e-10))

    cn2_col = jnp.sum(c * c, axis=1, keepdims=True)
    gram = jnp.dot(c, ct, preferred_element_type=jnp.float32)
    pd2 = cn2_col + cn2_row - 2.0 * gram
    pd = jnp.where(pd2 > 0, jnp.sqrt(jnp.where(pd2 > 0, pd2, 1.0)), 0.0)
    iota_r = jax.lax.broadcasted_iota(jnp.int32, (NUM_CLUSTERS, NUM_CLUSTERS), 0)
    iota_c = jax.lax.broadcasted_iota(jnp.int32, (NUM_CLUSTERS, NUM_CLUSTERS), 1)
    pd = pd + jnp.where(iota_r == iota_c, 1e10, 0.0)
    min_dist = -jnp.min(pd)

    counts = jnp.sum(partials_ref[...], axis=0, keepdims=True)

    total = jnp.sum(counts)
    proportions = counts / total
    gini = jnp.sum(proportions * (1.0 - proportions))
    mean_count = total * (1.0 / NUM_CLUSTERS)
    std_count = jnp.sqrt(jnp.mean((counts - mean_count) ** 2))
    imbalance = std_count / (mean_count + 1e-10)

    distance_loss = jnp.sum(colsum * cn_row)
    loss = (distance_loss - SINKHORN_REG * entropy + 0.2 * min_dist
            + 0.5 * gini + 0.8 * imbalance)
    loss_ref[0, 0] = loss


def kernel(proxy_points, centers, logits):
    del proxy_points
    ct = centers.T
    ltrw = logits.T.reshape(NUM_CLUSTERS, NUM_WORKERS,
                            ROWS_PER_WORKER).transpose(1, 0, 2)
    partials = _sc_hist(ct, ltrw)
    probs, loss = pl.pallas_call(
        _vwc_body,
        out_shape=(
            jax.ShapeDtypeStruct((NUM_CLIENTS, NUM_CLUSTERS), jnp.float32),
            jax.ShapeDtypeStruct((1, 1), jnp.float32),
        ),
        out_specs=(
            pl.BlockSpec(memory_space=pltpu.VMEM),
            pl.BlockSpec(memory_space=pltpu.SMEM),
        ),
        in_specs=(
            pl.BlockSpec(memory_space=pltpu.VMEM),
            pl.BlockSpec(memory_space=pltpu.VMEM),
            pl.BlockSpec(memory_space=pltpu.VMEM),
            pl.BlockSpec(memory_space=pltpu.VMEM),
        ),
    )(centers, ct, logits, partials)
    return probs, loss.reshape(())
